# Initial kernel scaffold; baseline (speedup 1.0000x reference)
import jax
import jax.numpy as jnp
from jax.experimental import pallas as pl


def kernel(x, edge_index, edge_attr, pos, batch, W1, r1, b1, W2, r2, b2, W3, r3, b3, fc1_w, fc1_b, fc2_w, fc2_b):
    raise NotImplementedError("write your pallas kernel here")


# speedup vs baseline: 2.9356x; 2.9356x over previous
import jax
import jax.numpy as jnp
from jax import lax
from jax.experimental import pallas as pl

NG = 1024
NPG = 75
EPG = 600
KK = 25
G1 = 36
G2 = 16
G3 = 4
F32 = jnp.float32
I32 = jnp.int32


def _elu(v):
    return jnp.where(v > 0, v, jnp.exp(jnp.minimum(v, 0.0)) - 1.0)


def _basis(ps, ev):
    scaled = ps * 4.0
    k0f = jnp.clip(jnp.floor(scaled), 0.0, 3.0)
    frac = scaled - k0f
    k0 = k0f.astype(I32)
    iota = lax.broadcasted_iota(I32, (EPG, KK), 1)
    B = jnp.zeros((EPG, KK), F32)
    for b0 in (0, 1):
        for b1 in (0, 1):
            w0 = frac[:, 0] if b0 else 1.0 - frac[:, 0]
            w1 = frac[:, 1] if b1 else 1.0 - frac[:, 1]
            coef = w0 * w1 * ev
            kidx = (k0[:, 0] + b0) * 5 + (k0[:, 1] + b1)
            B = B + jnp.where(kidx[:, None] == iota, coef[:, None], 0.0)
    return B


def _conv(hp, Sn, Dn, Bm, Wf, nev, r, b, dout):
    sx = jnp.dot(Sn, hp, preferred_element_type=F32)
    V = jnp.dot(sx, Wf, preferred_element_type=F32)
    msg = jnp.sum(V.reshape(EPG, KK, dout) * Bm[:, :, None], axis=1)
    acc = lax.dot_general(Dn, msg, (((0,), (0,)), ((), ())),
                          preferred_element_type=F32)
    deg = jnp.sum(Dn * nev[:, None], axis=0)
    out = acc / jnp.maximum(deg, 1.0)[:, None] + jnp.dot(hp, r) + b
    return _elu(out)


_BIG = 3.0e38


def _segmax(h, Cf):
    pen = (1.0 - Cf) * _BIG
    m = h[:, None, :] - pen[:, :, None]
    mx = jnp.max(m, axis=0)
    return jnp.where(mx > -0.5 * _BIG, mx, 0.0)


def _phase_a(x_ref, srcl_ref, dstl_ref, ea_ref, pos_ref, w_ref, r_ref, b_ref,
             h_ref, npos_ref, cnt_ref, nsrc_ref, ndst_ref, nev_ref, rel_ref, mx_ref):
    x = x_ref[0]
    srcl = srcl_ref[0, 0]
    dstl = dstl_ref[0, 0]
    ea = ea_ref[0]
    pos = pos_ref[0]
    io_n = lax.broadcasted_iota(I32, (EPG, NPG), 1)
    S = (srcl[:, None] == io_n).astype(F32)
    D = (dstl[:, None] == io_n).astype(F32)
    ones = jnp.ones((EPG,), F32)
    Bm = _basis(ea, ones)
    h = _conv(x, S, D, Bm, w_ref[...], ones, r_ref[...], b_ref[...], 32)
    c = jnp.clip(jnp.floor(pos * 0.2).astype(I32), 0, 5)
    cl = c[:, 0] * 6 + c[:, 1]
    io_c = lax.broadcasted_iota(I32, (NPG, G1), 1)
    Cf = (cl[:, None] == io_c).astype(F32)
    hp = _segmax(h, Cf)
    cnt = jnp.sum(Cf, axis=0)
    psum = lax.dot_general(Cf, pos, (((0,), (0,)), ((), ())),
                           preferred_element_type=F32)
    npos = psum / jnp.maximum(cnt, 1.0)[:, None]
    clf = cl.astype(F32)
    nsrc = jnp.sum(S * clf[None, :], axis=1).astype(I32)
    ndst = jnp.sum(D * clf[None, :], axis=1).astype(I32)
    nevf = (nsrc != ndst).astype(F32)
    io_e = lax.broadcasted_iota(I32, (EPG, G1), 1)
    Sn = (nsrc[:, None] == io_e).astype(F32)
    Dn = (ndst[:, None] == io_e).astype(F32)
    rel = jnp.dot(Dn, npos) - jnp.dot(Sn, npos)
    mx = jnp.max(jnp.abs(rel) * nevf[:, None])
    h_ref[0] = hp
    npos_ref[0] = npos
    cnt_ref[0, 0] = cnt
    nsrc_ref[0, 0] = nsrc
    ndst_ref[0, 0] = ndst
    nev_ref[0, 0] = nevf
    rel_ref[0] = rel
    mx_ref[0, 0] = mx.reshape((1,))


def _phase_b(h_ref, npos_ref, cnt_ref, nsrc_ref, ndst_ref, nev_ref, rel_ref,
             mxall_ref, w_ref, r_ref, b_ref,
             h2_ref, npos2_ref, cnt2_ref, nsrc2_ref, ndst2_ref, nev2_ref,
             rel2_ref, mx2_ref):
    md = jnp.maximum(jnp.max(mxall_ref[...]), 1e-8)
    rel = rel_ref[0]
    ps = jnp.clip(rel / (2.0 * md) + 0.5, 0.0, 1.0)
    nevf = nev_ref[0, 0]
    nsrc = nsrc_ref[0, 0]
    ndst = ndst_ref[0, 0]
    io_e = lax.broadcasted_iota(I32, (EPG, G1), 1)
    Sn = (nsrc[:, None] == io_e).astype(F32)
    Dn = (ndst[:, None] == io_e).astype(F32)
    Bm = _basis(ps, nevf)
    hp = h_ref[0]
    h2 = _conv(hp, Sn, Dn, Bm, w_ref[...], nevf, r_ref[...], b_ref[...], 64)
    npos1 = npos_ref[0]
    valid1f = jnp.where(cnt_ref[0, 0] > 0.5, 1.0, 0.0)
    c = jnp.clip(jnp.floor(npos1 / 7.0).astype(I32), 0, 3)
    cl = c[:, 0] * 4 + c[:, 1]
    io_c = lax.broadcasted_iota(I32, (G1, G2), 1)
    Cf = (cl[:, None] == io_c).astype(F32) * valid1f[:, None]
    hp2 = _segmax(h2, Cf)
    cnt2 = jnp.sum(Cf, axis=0)
    psum2 = lax.dot_general(Cf, npos1, (((0,), (0,)), ((), ())),
                            preferred_element_type=F32)
    npos2 = psum2 / jnp.maximum(cnt2, 1.0)[:, None]
    clf = cl.astype(F32)
    nsrc2 = jnp.sum(Sn * clf[None, :], axis=1).astype(I32)
    ndst2 = jnp.sum(Dn * clf[None, :], axis=1).astype(I32)
    nev2f = nevf * (nsrc2 != ndst2).astype(F32)
    io_e2 = lax.broadcasted_iota(I32, (EPG, G2), 1)
    Sn2 = (nsrc2[:, None] == io_e2).astype(F32)
    Dn2 = (ndst2[:, None] == io_e2).astype(F32)
    rel2 = jnp.dot(Dn2, npos2) - jnp.dot(Sn2, npos2)
    mx2 = jnp.max(jnp.abs(rel2) * nev2f[:, None])
    h2_ref[0] = hp2
    npos2_ref[0] = npos2
    cnt2_ref[0, 0] = cnt2
    nsrc2_ref[0, 0] = nsrc2
    ndst2_ref[0, 0] = ndst2
    nev2_ref[0, 0] = nev2f
    rel2_ref[0] = rel2
    mx2_ref[0, 0] = mx2.reshape((1,))


def _phase_c(h2_ref, npos2_ref, cnt2_ref, nsrc2_ref, ndst2_ref, nev2_ref,
             rel2_ref, mx2all_ref, w_ref, r_ref, b_ref, g_ref):
    md = jnp.maximum(jnp.max(mx2all_ref[...]), 1e-8)
    rel = rel2_ref[0]
    ps = jnp.clip(rel / (2.0 * md) + 0.5, 0.0, 1.0)
    nevf = nev2_ref[0, 0]
    nsrc = nsrc2_ref[0, 0]
    ndst = ndst2_ref[0, 0]
    io_e = lax.broadcasted_iota(I32, (EPG, G2), 1)
    Sn = (nsrc[:, None] == io_e).astype(F32)
    Dn = (ndst[:, None] == io_e).astype(F32)
    Bm = _basis(ps, nevf)
    hp2 = h2_ref[0]
    h3 = _conv(hp2, Sn, Dn, Bm, w_ref[...], nevf, r_ref[...], b_ref[...], 64)
    npos2 = npos2_ref[0]
    valid2f = jnp.where(cnt2_ref[0, 0] > 0.5, 1.0, 0.0)
    c = jnp.clip(jnp.floor(npos2 / 14.0).astype(I32), 0, 1)
    cl = c[:, 0] * 2 + c[:, 1]
    io_c = lax.broadcasted_iota(I32, (G2, G3), 1)
    Cf = (cl[:, None] == io_c).astype(F32) * valid2f[:, None]
    g_ref[0] = _segmax(h3, Cf)


def _phase_d(g_ref, w1_ref, b1_ref, w2_ref, b2_ref, o_ref):
    g = g_ref[...]
    a = _elu(jnp.dot(g, w1_ref[...], preferred_element_type=F32) + b1_ref[...])
    z = jnp.dot(a, w2_ref[...], preferred_element_type=F32) + b2_ref[...]
    zs = z - jnp.max(z, axis=1, keepdims=True)
    o_ref[...] = zs - jnp.log(jnp.sum(jnp.exp(zs), axis=1, keepdims=True))


def _spec(shape, imap):
    return pl.BlockSpec(shape, imap)


def _g3(g):
    return (g, 0, 0)


def _z2(g):
    return (0, 0)


def _z3(g):
    return (0, 0, 0)


def kernel(x, edge_index, edge_attr, pos, batch, W1, r1, b1, W2, r2, b2,
           W3, r3, b3, fc1_w, fc1_b, fc2_w, fc2_b):
    E = NG * EPG
    eoff = (jnp.arange(E, dtype=I32) // EPG) * NPG
    srcl = (edge_index[0].astype(I32) - eoff).reshape(NG, 1, EPG)
    dstl = (edge_index[1].astype(I32) - eoff).reshape(NG, 1, EPG)
    x3 = x.reshape(NG, NPG, 1)
    ea3 = edge_attr.reshape(NG, EPG, 2)
    pos3 = pos.reshape(NG, NPG, 2)
    w1f = W1.transpose(1, 0, 2).reshape(1, KK * 32)
    w2f = W2.transpose(1, 0, 2).reshape(32, KK * 64)
    w3f = W3.transpose(1, 0, 2).reshape(64, KK * 64)

    a_out = pl.pallas_call(
        _phase_a,
        grid=(NG,),
        in_specs=[
            _spec((1, NPG, 1), _g3),
            _spec((1, 1, EPG), _g3),
            _spec((1, 1, EPG), _g3),
            _spec((1, EPG, 2), _g3),
            _spec((1, NPG, 2), _g3),
            _spec((1, KK * 32), _z2),
            _spec((1, 32), _z2),
            _spec((1, 32), _z2),
        ],
        out_specs=[
            _spec((1, G1, 32), _g3),
            _spec((1, G1, 2), _g3),
            _spec((1, 1, G1), _g3),
            _spec((1, 1, EPG), _g3),
            _spec((1, 1, EPG), _g3),
            _spec((1, 1, EPG), _g3),
            _spec((1, EPG, 2), _g3),
            _spec((1, 1, 1), _g3),
        ],
        out_shape=[
            jax.ShapeDtypeStruct((NG, G1, 32), F32),
            jax.ShapeDtypeStruct((NG, G1, 2), F32),
            jax.ShapeDtypeStruct((NG, 1, G1), F32),
            jax.ShapeDtypeStruct((NG, 1, EPG), I32),
            jax.ShapeDtypeStruct((NG, 1, EPG), I32),
            jax.ShapeDtypeStruct((NG, 1, EPG), F32),
            jax.ShapeDtypeStruct((NG, EPG, 2), F32),
            jax.ShapeDtypeStruct((NG, 1, 1), F32),
        ],
    )(x3, srcl, dstl, ea3, pos3, w1f, r1, b1.reshape(1, 32))
    hp1, npos1, cnt1, nsrc1, ndst1, nev1, rel1, mx1 = a_out

    b_out = pl.pallas_call(
        _phase_b,
        grid=(NG,),
        in_specs=[
            _spec((1, G1, 32), _g3),
            _spec((1, G1, 2), _g3),
            _spec((1, 1, G1), _g3),
            _spec((1, 1, EPG), _g3),
            _spec((1, 1, EPG), _g3),
            _spec((1, 1, EPG), _g3),
            _spec((1, EPG, 2), _g3),
            _spec((1, NG), _z2),
            _spec((32, KK * 64), _z2),
            _spec((32, 64), _z2),
            _spec((1, 64), _z2),
        ],
        out_specs=[
            _spec((1, G2, 64), _g3),
            _spec((1, G2, 2), _g3),
            _spec((1, 1, G2), _g3),
            _spec((1, 1, EPG), _g3),
            _spec((1, 1, EPG), _g3),
            _spec((1, 1, EPG), _g3),
            _spec((1, EPG, 2), _g3),
            _spec((1, 1, 1), _g3),
        ],
        out_shape=[
            jax.ShapeDtypeStruct((NG, G2, 64), F32),
            jax.ShapeDtypeStruct((NG, G2, 2), F32),
            jax.ShapeDtypeStruct((NG, 1, G2), F32),
            jax.ShapeDtypeStruct((NG, 1, EPG), I32),
            jax.ShapeDtypeStruct((NG, 1, EPG), I32),
            jax.ShapeDtypeStruct((NG, 1, EPG), F32),
            jax.ShapeDtypeStruct((NG, EPG, 2), F32),
            jax.ShapeDtypeStruct((NG, 1, 1), F32),
        ],
    )(hp1, npos1, cnt1, nsrc1, ndst1, nev1, rel1, mx1.reshape(1, NG),
      w2f, r2, b2.reshape(1, 64))
    hp2, npos2, cnt2, nsrc2, ndst2, nev2, rel2, mx2 = b_out

    g4 = pl.pallas_call(
        _phase_c,
        grid=(NG,),
        in_specs=[
            _spec((1, G2, 64), _g3),
            _spec((1, G2, 2), _g3),
            _spec((1, 1, G2), _g3),
            _spec((1, 1, EPG), _g3),
            _spec((1, 1, EPG), _g3),
            _spec((1, 1, EPG), _g3),
            _spec((1, EPG, 2), _g3),
            _spec((1, NG), _z2),
            _spec((64, KK * 64), _z2),
            _spec((64, 64), _z2),
            _spec((1, 64), _z2),
        ],
        out_specs=[_spec((1, G3, 64), _g3)],
        out_shape=[jax.ShapeDtypeStruct((NG, G3, 64), F32)],
    )(hp2, npos2, cnt2, nsrc2, ndst2, nev2, rel2, mx2.reshape(1, NG),
      w3f, r3, b3.reshape(1, 64))[0]

    out = pl.pallas_call(
        _phase_d,
        grid=(1,),
        in_specs=[
            _spec((NG, G3 * 64), lambda i: (0, 0)),
            _spec((256, 128), lambda i: (0, 0)),
            _spec((1, 128), lambda i: (0, 0)),
            _spec((128, 10), lambda i: (0, 0)),
            _spec((1, 10), lambda i: (0, 0)),
        ],
        out_specs=pl.BlockSpec((NG, 10), lambda i: (0, 0)),
        out_shape=jax.ShapeDtypeStruct((NG, 10), F32),
    )(g4.reshape(NG, G3 * 64), fc1_w, fc1_b.reshape(1, 128),
      fc2_w, fc2_b.reshape(1, 10))
    return out


# speedup vs baseline: 6.0132x; 2.0484x over previous
import jax
import jax.numpy as jnp
from jax import lax
from jax.experimental import pallas as pl
from jax.experimental.pallas import tpu as pltpu

NG = 1024
NPG = 75
EPG = 600
KK = 25
KP = 32
G1 = 36
G2 = 16
G3 = 4
F32 = jnp.float32
BF16 = jnp.bfloat16
I32 = jnp.int32
_BIG = 3.0e38


def _elu(v):
    return jnp.where(v > 0, v, jnp.exp(jnp.minimum(v, 0.0)) - 1.0)


def _basis(ps, ev):
    scaled = ps * 4.0
    k0f = jnp.clip(jnp.floor(scaled), 0.0, 3.0)
    frac = scaled - k0f
    iof = lax.broadcasted_iota(I32, (EPG, KP), 1).astype(F32)
    B = jnp.zeros((EPG, KP), F32)
    for b0 in (0, 1):
        for b1 in (0, 1):
            w0 = frac[:, 0:1] if b0 else 1.0 - frac[:, 0:1]
            w1 = frac[:, 1:2] if b1 else 1.0 - frac[:, 1:2]
            coef = w0 * w1 * ev
            kidx = (k0f[:, 0:1] + b0) * 5.0 + (k0f[:, 1:2] + b1)
            B = B + jnp.where(kidx == iof, coef, 0.0)
    return B


def _conv(hp, Sn, Dn, Bm, X, Wkr, nev, r, b):
    din = hp.shape[1]
    sx = jnp.dot(Sn, hp, preferred_element_type=F32)
    sxt = pltpu.repeat(sx, KK, axis=1)
    Bt = jnp.dot(Bm.astype(BF16), X, preferred_element_type=F32)
    msg = jnp.dot((Bt * sxt).astype(BF16), Wkr,
                  preferred_element_type=F32)
    acc = lax.dot_general(Dn, msg, (((0,), (0,)), ((), ())),
                          preferred_element_type=F32)
    deg = lax.dot_general(Dn, nev, (((0,), (0,)), ((), ())),
                          preferred_element_type=F32)
    out = acc / jnp.maximum(deg, 1.0) + jnp.dot(hp, r) + b
    return _elu(out)


def _segmax(h, Cf):
    pen = (1.0 - Cf) * _BIG
    m = h[:, None, :] - pen[:, :, None]
    mx = jnp.max(m, axis=0)
    return jnp.where(mx > -0.5 * _BIG, mx, 0.0)


def _colsum(Cf, v):
    return lax.dot_general(Cf, v, (((0,), (0,)), ((), ())),
                           preferred_element_type=F32)


def _phase_a(x_ref, srcl_ref, dstl_ref, ea_ref, pos_ref, w_ref, r_ref, b_ref,
             h_ref, npos_ref, cnt_ref, nsrc_ref, ndst_ref, nev_ref, rel_ref,
             mx_ref):
    x = x_ref[0]
    srcl = srcl_ref[0]
    dstl = dstl_ref[0]
    ea = ea_ref[0]
    pos = pos_ref[0]
    io_n = lax.broadcasted_iota(I32, (EPG, NPG), 1).astype(F32)
    S = (srcl == io_n).astype(F32)
    D = (dstl == io_n).astype(F32)
    ones = jnp.ones((EPG, 1), F32)
    Bm = _basis(ea, ones)
    sx = jnp.dot(S, x, preferred_element_type=F32)
    prod = Bm[:, :KK] * sx
    msg = jnp.dot(prod.astype(BF16), w_ref[...],
                  preferred_element_type=F32)
    acc = _colsum(D, msg)
    deg = _colsum(D, ones)
    h = _elu(acc / jnp.maximum(deg, 1.0) + jnp.dot(x, r_ref[...]) + b_ref[...])
    c = jnp.clip(jnp.floor(pos * 0.2), 0.0, 5.0)
    cl = c[:, 0:1] * 6.0 + c[:, 1:2]
    io_c = lax.broadcasted_iota(I32, (NPG, G1), 1).astype(F32)
    Cf = (cl == io_c).astype(F32)
    hp = _segmax(h[:, :32], Cf)
    cnt = _colsum(Cf, jnp.ones((NPG, 1), F32))
    npos = _colsum(Cf, pos) / jnp.maximum(cnt, 1.0)
    nsrc = jnp.dot(S, cl, preferred_element_type=F32)
    ndst = jnp.dot(D, cl, preferred_element_type=F32)
    nevf = (nsrc != ndst).astype(F32)
    io_e = lax.broadcasted_iota(I32, (EPG, G1), 1).astype(F32)
    Sn = (nsrc == io_e).astype(F32)
    Dn = (ndst == io_e).astype(F32)
    rel = jnp.dot(Dn, npos) - jnp.dot(Sn, npos)
    mx = jnp.max(jnp.abs(rel) * nevf)
    h_ref[0] = hp
    npos_ref[0] = npos
    cnt_ref[0] = cnt
    nsrc_ref[0] = nsrc
    ndst_ref[0] = ndst
    nev_ref[0] = nevf
    rel_ref[0] = rel
    mx_ref[0, 0] = mx.reshape((1,))


def _phase_b(h_ref, npos_ref, cnt_ref, nsrc_ref, ndst_ref, nev_ref, rel_ref,
             mxall_ref, x_ref, w_ref, r_ref, b_ref,
             h2_ref, npos2_ref, cnt2_ref, nsrc2_ref, ndst2_ref, nev2_ref,
             rel2_ref, mx2_ref):
    md = jnp.maximum(jnp.max(mxall_ref[...]), 1e-8)
    rel = rel_ref[0]
    ps = jnp.clip(rel / (2.0 * md) + 0.5, 0.0, 1.0)
    nevf = nev_ref[0]
    nsrc = nsrc_ref[0]
    ndst = ndst_ref[0]
    io_e = lax.broadcasted_iota(I32, (EPG, G1), 1).astype(F32)
    Sn = (nsrc == io_e).astype(F32)
    Dn = (ndst == io_e).astype(F32)
    Bm = _basis(ps, nevf)
    hp = h_ref[0]
    h2 = _conv(hp, Sn, Dn, Bm, x_ref[...], w_ref[...], nevf,
               r_ref[...], b_ref[...])
    npos1 = npos_ref[0]
    validf = jnp.where(cnt_ref[0] > 0.5, 1.0, 0.0)
    c = jnp.clip(jnp.floor(npos1 / 7.0), 0.0, 3.0)
    cl = c[:, 0:1] * 4.0 + c[:, 1:2]
    io_c = lax.broadcasted_iota(I32, (G1, G2), 1).astype(F32)
    Cf = (cl == io_c).astype(F32) * validf
    hp2 = _segmax(h2[:, :64], Cf)
    cnt2 = _colsum(Cf, jnp.ones((G1, 1), F32))
    npos2 = _colsum(Cf, npos1) / jnp.maximum(cnt2, 1.0)
    nsrc2 = jnp.dot(Sn, cl, preferred_element_type=F32)
    ndst2 = jnp.dot(Dn, cl, preferred_element_type=F32)
    nev2f = nevf * (nsrc2 != ndst2).astype(F32)
    io_e2 = lax.broadcasted_iota(I32, (EPG, G2), 1).astype(F32)
    Sn2 = (nsrc2 == io_e2).astype(F32)
    Dn2 = (ndst2 == io_e2).astype(F32)
    rel2 = jnp.dot(Dn2, npos2) - jnp.dot(Sn2, npos2)
    mx2 = jnp.max(jnp.abs(rel2) * nev2f)
    h2_ref[0] = hp2
    npos2_ref[0] = npos2
    cnt2_ref[0] = cnt2
    nsrc2_ref[0] = nsrc2
    ndst2_ref[0] = ndst2
    nev2_ref[0] = nev2f
    rel2_ref[0] = rel2
    mx2_ref[0, 0] = mx2.reshape((1,))


def _phase_c(h2_ref, npos2_ref, cnt2_ref, nsrc2_ref, ndst2_ref, nev2_ref,
             rel2_ref, mx2all_ref, x_ref, w_ref, r_ref, b_ref, g_ref):
    md = jnp.maximum(jnp.max(mx2all_ref[...]), 1e-8)
    rel = rel2_ref[0]
    ps = jnp.clip(rel / (2.0 * md) + 0.5, 0.0, 1.0)
    nevf = nev2_ref[0]
    nsrc = nsrc2_ref[0]
    ndst = ndst2_ref[0]
    io_e = lax.broadcasted_iota(I32, (EPG, G2), 1).astype(F32)
    Sn = (nsrc == io_e).astype(F32)
    Dn = (ndst == io_e).astype(F32)
    Bm = _basis(ps, nevf)
    hp2 = h2_ref[0]
    h3 = _conv(hp2, Sn, Dn, Bm, x_ref[...], w_ref[...], nevf,
               r_ref[...], b_ref[...])
    npos2 = npos2_ref[0]
    valid2f = jnp.where(cnt2_ref[0] > 0.5, 1.0, 0.0)
    c = jnp.clip(jnp.floor(npos2 / 14.0), 0.0, 1.0)
    cl = c[:, 0:1] * 2.0 + c[:, 1:2]
    io_c = lax.broadcasted_iota(I32, (G2, G3), 1).astype(F32)
    Cf = (cl == io_c).astype(F32) * valid2f
    g_ref[0] = _segmax(h3[:, :64], Cf)


def _phase_d(g_ref, w1_ref, b1_ref, w2_ref, b2_ref, o_ref):
    g = g_ref[...]
    a = _elu(jnp.dot(g, w1_ref[...], preferred_element_type=F32) + b1_ref[...])
    z = jnp.dot(a, w2_ref[...], preferred_element_type=F32) + b2_ref[...]
    zs = z - jnp.max(z, axis=1, keepdims=True)
    o_ref[...] = zs - jnp.log(jnp.sum(jnp.exp(zs), axis=1, keepdims=True))


def _spec(shape, imap):
    return pl.BlockSpec(shape, imap)


def _g3(g):
    return (g, 0, 0)


def _z2(g):
    return (0, 0)


def _expander(din):
    row = jnp.arange(KP, dtype=I32)[:, None]
    col = jnp.arange(KK * din, dtype=I32)[None, :]
    return ((col // din == row) & (row < KK)).astype(BF16)


def _padn(w, n=128):
    return jnp.pad(w, ((0, 0), (0, n - w.shape[1])))


def kernel(x, edge_index, edge_attr, pos, batch, W1, r1, b1, W2, r2, b2,
           W3, r3, b3, fc1_w, fc1_b, fc2_w, fc2_b):
    E = NG * EPG
    eoff = (jnp.arange(E, dtype=I32) // EPG) * NPG
    srcl = (edge_index[0].astype(I32) - eoff).astype(F32).reshape(NG, EPG, 1)
    dstl = (edge_index[1].astype(I32) - eoff).astype(F32).reshape(NG, EPG, 1)
    x3 = x.reshape(NG, NPG, 1)
    ea3 = edge_attr.reshape(NG, EPG, 2)
    pos3 = pos.reshape(NG, NPG, 2)
    w1p = _padn(W1.reshape(KK, 32)).astype(BF16)
    w2p = _padn(W2.reshape(KK * 32, 64)).astype(BF16)
    w3p = _padn(W3.reshape(KK * 64, 64)).astype(BF16)
    r1p = _padn(r1)
    r2p = _padn(r2)
    r3p = _padn(r3)
    b1p = _padn(b1.reshape(1, 32))
    b2p = _padn(b2.reshape(1, 64))
    b3p = _padn(b3.reshape(1, 64))
    x2e = _expander(32)
    x3e = _expander(64)

    ecol = lambda: _spec((1, EPG, 1), _g3)
    a_out = pl.pallas_call(
        _phase_a,
        grid=(NG,),
        in_specs=[
            _spec((1, NPG, 1), _g3), ecol(), ecol(),
            _spec((1, EPG, 2), _g3),
            _spec((1, NPG, 2), _g3),
            _spec((KK, 128), _z2),
            _spec((1, 128), _z2),
            _spec((1, 128), _z2),
        ],
        out_specs=[
            _spec((1, G1, 32), _g3),
            _spec((1, G1, 2), _g3),
            _spec((1, G1, 1), _g3),
            ecol(), ecol(), ecol(),
            _spec((1, EPG, 2), _g3),
            _spec((1, 1, 1), _g3),
        ],
        out_shape=[
            jax.ShapeDtypeStruct((NG, G1, 32), F32),
            jax.ShapeDtypeStruct((NG, G1, 2), F32),
            jax.ShapeDtypeStruct((NG, G1, 1), F32),
            jax.ShapeDtypeStruct((NG, EPG, 1), F32),
            jax.ShapeDtypeStruct((NG, EPG, 1), F32),
            jax.ShapeDtypeStruct((NG, EPG, 1), F32),
            jax.ShapeDtypeStruct((NG, EPG, 2), F32),
            jax.ShapeDtypeStruct((NG, 1, 1), F32),
        ],
    )(x3, srcl, dstl, ea3, pos3, w1p, r1p, b1p)
    hp1, npos1, cnt1, nsrc1, ndst1, nev1, rel1, mx1 = a_out

    b_out = pl.pallas_call(
        _phase_b,
        grid=(NG,),
        in_specs=[
            _spec((1, G1, 32), _g3),
            _spec((1, G1, 2), _g3),
            _spec((1, G1, 1), _g3),
            ecol(), ecol(), ecol(),
            _spec((1, EPG, 2), _g3),
            _spec((1, NG), _z2),
            _spec((KP, KK * 32), _z2),
            _spec((KK * 32, 128), _z2),
            _spec((32, 128), _z2),
            _spec((1, 128), _z2),
        ],
        out_specs=[
            _spec((1, G2, 64), _g3),
            _spec((1, G2, 2), _g3),
            _spec((1, G2, 1), _g3),
            ecol(), ecol(), ecol(),
            _spec((1, EPG, 2), _g3),
            _spec((1, 1, 1), _g3),
        ],
        out_shape=[
            jax.ShapeDtypeStruct((NG, G2, 64), F32),
            jax.ShapeDtypeStruct((NG, G2, 2), F32),
            jax.ShapeDtypeStruct((NG, G2, 1), F32),
            jax.ShapeDtypeStruct((NG, EPG, 1), F32),
            jax.ShapeDtypeStruct((NG, EPG, 1), F32),
            jax.ShapeDtypeStruct((NG, EPG, 1), F32),
            jax.ShapeDtypeStruct((NG, EPG, 2), F32),
            jax.ShapeDtypeStruct((NG, 1, 1), F32),
        ],
    )(hp1, npos1, cnt1, nsrc1, ndst1, nev1, rel1, mx1.reshape(1, NG),
      x2e, w2p, r2p, b2p)
    hp2, npos2, cnt2, nsrc2, ndst2, nev2, rel2, mx2 = b_out

    g4 = pl.pallas_call(
        _phase_c,
        grid=(NG,),
        in_specs=[
            _spec((1, G2, 64), _g3),
            _spec((1, G2, 2), _g3),
            _spec((1, G2, 1), _g3),
            ecol(), ecol(), ecol(),
            _spec((1, EPG, 2), _g3),
            _spec((1, NG), _z2),
            _spec((KP, KK * 64), _z2),
            _spec((KK * 64, 128), _z2),
            _spec((64, 128), _z2),
            _spec((1, 128), _z2),
        ],
        out_specs=[_spec((1, G3, 64), _g3)],
        out_shape=[jax.ShapeDtypeStruct((NG, G3, 64), F32)],
    )(hp2, npos2, cnt2, nsrc2, ndst2, nev2, rel2, mx2.reshape(1, NG),
      x3e, w3p, r3p, b3p)[0]

    out = pl.pallas_call(
        _phase_d,
        grid=(1,),
        in_specs=[
            _spec((NG, G3 * 64), lambda i: (0, 0)),
            _spec((256, 128), lambda i: (0, 0)),
            _spec((1, 128), lambda i: (0, 0)),
            _spec((128, 10), lambda i: (0, 0)),
            _spec((1, 10), lambda i: (0, 0)),
        ],
        out_specs=pl.BlockSpec((NG, 10), lambda i: (0, 0)),
        out_shape=jax.ShapeDtypeStruct((NG, 10), F32),
    )(g4.reshape(NG, G3 * 64), fc1_w, fc1_b.reshape(1, 128),
      fc2_w, fc2_b.reshape(1, 10))
    return out


# speedup vs baseline: 12.7816x; 2.1256x over previous
import jax
import jax.numpy as jnp
from jax import lax
from jax.experimental import pallas as pl
from jax.experimental.pallas import tpu as pltpu

NG = 1024
NPG = 75
EPG = 600
KK = 25
KP = 32
G1 = 36
G2 = 16
G3 = 4
F32 = jnp.float32
BF16 = jnp.bfloat16
I32 = jnp.int32
_BIG = 3.0e38


def _elu(v):
    return jnp.where(v > 0, v, jnp.exp(jnp.minimum(v, 0.0)) - 1.0)


def _dot0(a, b):
    return lax.dot_general(a, b, (((0,), (0,)), ((), ())),
                           preferred_element_type=F32)


def _basis_t(ps, ev):
    scaled = ps * 4.0
    k0f = jnp.clip(jnp.floor(scaled), 0.0, 3.0)
    frac = scaled - k0f
    iof = lax.broadcasted_iota(I32, (KP, EPG), 0).astype(F32)
    B = jnp.zeros((KP, EPG), F32)
    for b0 in (0, 1):
        for b1 in (0, 1):
            w0 = frac[0:1] if b0 else 1.0 - frac[0:1]
            w1 = frac[1:2] if b1 else 1.0 - frac[1:2]
            coef = w0 * w1 * ev
            kidx = (k0f[0:1] + b0) * 5.0 + (k0f[1:2] + b1)
            B = B + jnp.where(kidx == iof, coef, 0.0)
    return B


def _onehot_t(ids_row, n):
    io = lax.broadcasted_iota(I32, (n, EPG), 0).astype(F32)
    return (ids_row == io).astype(F32)


def _conv(hp, SnT, DnT, BmT, X, Wkr, nev, r, b):
    sx = _dot0(SnT, hp)
    sxt = pltpu.repeat(sx, KK, axis=1)
    Bt = _dot0(BmT.astype(BF16), X)
    msg = jnp.dot((Bt * sxt).astype(BF16), Wkr,
                  preferred_element_type=F32)
    acc = jnp.dot(DnT, msg, preferred_element_type=F32)
    deg = jnp.sum(DnT * nev, axis=1, keepdims=True)
    out = acc / jnp.maximum(deg, 1.0) + jnp.dot(hp, r) + b
    return _elu(out)


def _segmax(h, Cf):
    pen = (1.0 - Cf) * _BIG
    m = h[:, None, :] - pen[:, :, None]
    mx = jnp.max(m, axis=0)
    return jnp.where(mx > -0.5 * _BIG, mx, 0.0)


def _phase_a(x_ref, srcl_ref, dstl_ref, ea_ref, pos_ref, w_ref, r_ref, b_ref,
             h_ref, npos_ref, cnt_ref, nsrc_ref, ndst_ref, nev_ref, rel_ref,
             mx_ref):
    x = x_ref[0]
    srcl = srcl_ref[0]
    dstl = dstl_ref[0]
    ea = ea_ref[0]
    pos = pos_ref[0]
    ST = _onehot_t(srcl, NPG)
    DT = _onehot_t(dstl, NPG)
    ones = jnp.ones((1, EPG), F32)
    BmT = _basis_t(ea, ones)
    sx = _dot0(x, ST)
    prod = BmT[:KK] * sx
    msg = _dot0(prod.astype(BF16), w_ref[...])
    acc = jnp.dot(DT, msg, preferred_element_type=F32)
    deg = jnp.sum(DT, axis=1, keepdims=True)
    h = _elu(acc / jnp.maximum(deg, 1.0) + jnp.dot(x, r_ref[...]) + b_ref[...])
    c = jnp.clip(jnp.floor(pos * 0.2), 0.0, 5.0)
    cl = c[:, 0:1] * 6.0 + c[:, 1:2]
    io_c = lax.broadcasted_iota(I32, (NPG, G1), 1).astype(F32)
    Cf = (cl == io_c).astype(F32)
    hp = _segmax(h[:, :32], Cf)
    cnt = jnp.sum(Cf, axis=0, keepdims=True)
    npos = _dot0(Cf, pos) / jnp.maximum(cnt, 1.0).T
    nsrc = _dot0(cl, ST)
    ndst = _dot0(cl, DT)
    nevf = (nsrc != ndst).astype(F32)
    SnT = _onehot_t(nsrc, G1)
    DnT = _onehot_t(ndst, G1)
    rel = _dot0(npos, DnT) - _dot0(npos, SnT)
    mx = jnp.max(jnp.abs(rel) * nevf)
    h_ref[0] = hp
    npos_ref[0] = npos
    cnt_ref[0] = cnt
    nsrc_ref[0] = nsrc
    ndst_ref[0] = ndst
    nev_ref[0] = nevf
    rel_ref[0] = rel
    mx_ref[0, 0] = mx.reshape((1,))


def _phase_b(h_ref, npos_ref, cnt_ref, nsrc_ref, ndst_ref, nev_ref, rel_ref,
             mxall_ref, x_ref, w_ref, r_ref, b_ref,
             h2_ref, npos2_ref, cnt2_ref, nsrc2_ref, ndst2_ref, nev2_ref,
             rel2_ref, mx2_ref):
    md = jnp.maximum(jnp.max(mxall_ref[...]), 1e-8)
    rel = rel_ref[0]
    ps = jnp.clip(rel / (2.0 * md) + 0.5, 0.0, 1.0)
    nevf = nev_ref[0]
    SnT = _onehot_t(nsrc_ref[0], G1)
    DnT = _onehot_t(ndst_ref[0], G1)
    BmT = _basis_t(ps, nevf)
    hp = h_ref[0]
    h2 = _conv(hp, SnT, DnT, BmT, x_ref[...], w_ref[...], nevf,
               r_ref[...], b_ref[...])
    npos1 = npos_ref[0]
    validf = jnp.where(cnt_ref[0] > 0.5, 1.0, 0.0)
    c = jnp.clip(jnp.floor(npos1 / 7.0), 0.0, 3.0)
    cl = c[:, 0:1] * 4.0 + c[:, 1:2]
    io_c = lax.broadcasted_iota(I32, (G1, G2), 1).astype(F32)
    Cf = (cl == io_c).astype(F32) * validf.T
    hp2 = _segmax(h2[:, :64], Cf)
    cnt2 = jnp.sum(Cf, axis=0, keepdims=True)
    npos2 = _dot0(Cf, npos1) / jnp.maximum(cnt2, 1.0).T
    nsrc2 = _dot0(cl, SnT)
    ndst2 = _dot0(cl, DnT)
    nev2f = nevf * (nsrc2 != ndst2).astype(F32)
    Sn2T = _onehot_t(nsrc2, G2)
    Dn2T = _onehot_t(ndst2, G2)
    rel2 = _dot0(npos2, Dn2T) - _dot0(npos2, Sn2T)
    mx2 = jnp.max(jnp.abs(rel2) * nev2f)
    h2_ref[0] = hp2
    npos2_ref[0] = npos2
    cnt2_ref[0] = cnt2
    nsrc2_ref[0] = nsrc2
    ndst2_ref[0] = ndst2
    nev2_ref[0] = nev2f
    rel2_ref[0] = rel2
    mx2_ref[0, 0] = mx2.reshape((1,))


def _phase_c(h2_ref, npos2_ref, cnt2_ref, nsrc2_ref, ndst2_ref, nev2_ref,
             rel2_ref, mx2all_ref, x_ref, w_ref, r_ref, b_ref, g_ref):
    md = jnp.maximum(jnp.max(mx2all_ref[...]), 1e-8)
    rel = rel2_ref[0]
    ps = jnp.clip(rel / (2.0 * md) + 0.5, 0.0, 1.0)
    nevf = nev2_ref[0]
    SnT = _onehot_t(nsrc2_ref[0], G2)
    DnT = _onehot_t(ndst2_ref[0], G2)
    BmT = _basis_t(ps, nevf)
    hp2 = h2_ref[0]
    h3 = _conv(hp2, SnT, DnT, BmT, x_ref[...], w_ref[...], nevf,
               r_ref[...], b_ref[...])
    npos2 = npos2_ref[0]
    valid2f = jnp.where(cnt2_ref[0] > 0.5, 1.0, 0.0)
    c = jnp.clip(jnp.floor(npos2 / 14.0), 0.0, 1.0)
    cl = c[:, 0:1] * 2.0 + c[:, 1:2]
    io_c = lax.broadcasted_iota(I32, (G2, G3), 1).astype(F32)
    Cf = (cl == io_c).astype(F32) * valid2f.T
    g_ref[0] = _segmax(h3[:, :64], Cf)


def _phase_d(g_ref, w1_ref, b1_ref, w2_ref, b2_ref, o_ref):
    g = g_ref[...]
    a = _elu(jnp.dot(g, w1_ref[...], preferred_element_type=F32) + b1_ref[...])
    z = jnp.dot(a, w2_ref[...], preferred_element_type=F32) + b2_ref[...]
    zs = z - jnp.max(z, axis=1, keepdims=True)
    o_ref[...] = zs - jnp.log(jnp.sum(jnp.exp(zs), axis=1, keepdims=True))


def _spec(shape, imap):
    return pl.BlockSpec(shape, imap)


def _g3(g):
    return (g, 0, 0)


def _z2(g):
    return (0, 0)


def _expander(din):
    row = jnp.arange(KP, dtype=I32)[:, None]
    col = jnp.arange(KK * din, dtype=I32)[None, :]
    return ((col // din == row) & (row < KK)).astype(BF16)


def _padn(w, n=128):
    return jnp.pad(w, ((0, 0), (0, n - w.shape[1])))


def kernel(x, edge_index, edge_attr, pos, batch, W1, r1, b1, W2, r2, b2,
           W3, r3, b3, fc1_w, fc1_b, fc2_w, fc2_b):
    E = NG * EPG
    eoff = (jnp.arange(E, dtype=I32) // EPG) * NPG
    srcl = (edge_index[0].astype(I32) - eoff).astype(F32).reshape(NG, 1, EPG)
    dstl = (edge_index[1].astype(I32) - eoff).astype(F32).reshape(NG, 1, EPG)
    x3 = x.reshape(NG, NPG, 1)
    ea3 = edge_attr.reshape(NG, EPG, 2).transpose(0, 2, 1)
    pos3 = pos.reshape(NG, NPG, 2)
    w1p = _padn(W1.reshape(KK, 32)).astype(BF16)
    w2p = _padn(W2.reshape(KK * 32, 64)).astype(BF16)
    w3p = _padn(W3.reshape(KK * 64, 64)).astype(BF16)
    r1p = _padn(r1)
    r2p = _padn(r2)
    r3p = _padn(r3)
    b1p = _padn(b1.reshape(1, 32))
    b2p = _padn(b2.reshape(1, 64))
    b3p = _padn(b3.reshape(1, 64))
    x2e = _expander(32)
    x3e = _expander(64)

    erow = lambda: _spec((1, 1, EPG), _g3)
    e2row = lambda: _spec((1, 2, EPG), _g3)
    a_out = pl.pallas_call(
        _phase_a,
        grid=(NG,),
        in_specs=[
            _spec((1, NPG, 1), _g3), erow(), erow(), e2row(),
            _spec((1, NPG, 2), _g3),
            _spec((KK, 128), _z2),
            _spec((1, 128), _z2),
            _spec((1, 128), _z2),
        ],
        out_specs=[
            _spec((1, G1, 32), _g3),
            _spec((1, G1, 2), _g3),
            _spec((1, 1, G1), _g3),
            erow(), erow(), erow(), e2row(),
            _spec((1, 1, 1), _g3),
        ],
        out_shape=[
            jax.ShapeDtypeStruct((NG, G1, 32), F32),
            jax.ShapeDtypeStruct((NG, G1, 2), F32),
            jax.ShapeDtypeStruct((NG, 1, G1), F32),
            jax.ShapeDtypeStruct((NG, 1, EPG), F32),
            jax.ShapeDtypeStruct((NG, 1, EPG), F32),
            jax.ShapeDtypeStruct((NG, 1, EPG), F32),
            jax.ShapeDtypeStruct((NG, 2, EPG), F32),
            jax.ShapeDtypeStruct((NG, 1, 1), F32),
        ],
    )(x3, srcl, dstl, ea3, pos3, w1p, r1p, b1p)
    hp1, npos1, cnt1, nsrc1, ndst1, nev1, rel1, mx1 = a_out

    b_out = pl.pallas_call(
        _phase_b,
        grid=(NG,),
        in_specs=[
            _spec((1, G1, 32), _g3),
            _spec((1, G1, 2), _g3),
            _spec((1, 1, G1), _g3),
            erow(), erow(), erow(), e2row(),
            _spec((1, NG), _z2),
            _spec((KP, KK * 32), _z2),
            _spec((KK * 32, 128), _z2),
            _spec((32, 128), _z2),
            _spec((1, 128), _z2),
        ],
        out_specs=[
            _spec((1, G2, 64), _g3),
            _spec((1, G2, 2), _g3),
            _spec((1, 1, G2), _g3),
            erow(), erow(), erow(), e2row(),
            _spec((1, 1, 1), _g3),
        ],
        out_shape=[
            jax.ShapeDtypeStruct((NG, G2, 64), F32),
            jax.ShapeDtypeStruct((NG, G2, 2), F32),
            jax.ShapeDtypeStruct((NG, 1, G2), F32),
            jax.ShapeDtypeStruct((NG, 1, EPG), F32),
            jax.ShapeDtypeStruct((NG, 1, EPG), F32),
            jax.ShapeDtypeStruct((NG, 1, EPG), F32),
            jax.ShapeDtypeStruct((NG, 2, EPG), F32),
            jax.ShapeDtypeStruct((NG, 1, 1), F32),
        ],
    )(hp1, npos1, cnt1, nsrc1, ndst1, nev1, rel1, mx1.reshape(1, NG),
      x2e, w2p, r2p, b2p)
    hp2, npos2, cnt2, nsrc2, ndst2, nev2, rel2, mx2 = b_out

    g4 = pl.pallas_call(
        _phase_c,
        grid=(NG,),
        in_specs=[
            _spec((1, G2, 64), _g3),
            _spec((1, G2, 2), _g3),
            _spec((1, 1, G2), _g3),
            erow(), erow(), erow(), e2row(),
            _spec((1, NG), _z2),
            _spec((KP, KK * 64), _z2),
            _spec((KK * 64, 128), _z2),
            _spec((64, 128), _z2),
            _spec((1, 128), _z2),
        ],
        out_specs=[_spec((1, G3, 64), _g3)],
        out_shape=[jax.ShapeDtypeStruct((NG, G3, 64), F32)],
    )(hp2, npos2, cnt2, nsrc2, ndst2, nev2, rel2, mx2.reshape(1, NG),
      x3e, w3p, r3p, b3p)[0]

    out = pl.pallas_call(
        _phase_d,
        grid=(1,),
        in_specs=[
            _spec((NG, G3 * 64), lambda i: (0, 0)),
            _spec((256, 128), lambda i: (0, 0)),
            _spec((1, 128), lambda i: (0, 0)),
            _spec((128, 10), lambda i: (0, 0)),
            _spec((1, 10), lambda i: (0, 0)),
        ],
        out_specs=pl.BlockSpec((NG, 10), lambda i: (0, 0)),
        out_shape=jax.ShapeDtypeStruct((NG, 10), F32),
    )(g4.reshape(NG, G3 * 64), fc1_w, fc1_b.reshape(1, 128),
      fc2_w, fc2_b.reshape(1, 10))
    return out
